# baseline (speedup 1.0000x reference)
import functools

import jax
import jax.numpy as jnp
from jax import lax
from jax.experimental import pallas as pl
from jax.experimental.pallas import tpu as pltpu
from jax.experimental.pallas import tpu_sc as plsc

K = 10
PENAL = 0.05

B_BLK = 256
N_BLK = 2048
GRP = 16
NWORKERS = 32


def _mm_body(x_ref, w_ref, b_ref, h_ref, g_ref, inv_ref, sel_ref, *, n_blk):
    j = pl.program_id(0)
    i = pl.program_id(1)

    w = w_ref[...]

    @pl.when(i == 0)
    def _():
        ssq = jnp.sum(w * w, axis=1)
        inv_ref[0, pl.ds(j * n_blk, n_blk)] = 1.0 / (jnp.sqrt(ssq) + 1e-12)

    @pl.when((i == 0) & (j == 0))
    def _():
        rows = lax.broadcasted_iota(jnp.int32, (n_blk, n_blk // GRP), 0)
        cols = lax.broadcasted_iota(jnp.int32, (n_blk, n_blk // GRP), 1)
        sel_ref[...] = (rows == GRP * cols + (GRP - 1)).astype(jnp.float32)

    inv = inv_ref[0, pl.ds(j * n_blk, n_blk)]
    pre = lax.dot_general(
        x_ref[...], w,
        dimension_numbers=(((1,), (1,)), ((), ())),
        preferred_element_type=jnp.float32,
    ) + b_ref[0, :][None, :]
    h = jnp.maximum(PENAL - jnp.abs(pre) * inv[None, :], 0.0)
    h_ref[...] = h
    m = h
    for s in (1, 2, 4, 8):
        m = jnp.maximum(m, pltpu.roll(m, s, 1))
    g_ref[...] = lax.dot_general(
        m, sel_ref[...],
        dimension_numbers=(((1,), (0,)), ((), ())),
        preferred_element_type=jnp.float32,
    )


def _sc_body(gmax_hbm, hflat_hbm, out_hbm, gm_v, idx_v, gat_v, res_v, sem,
             *, rpw, ngrp):
    nc = 2
    wid = lax.axis_index("s") * nc + lax.axis_index("c")
    base = wid * rpw
    pltpu.sync_copy(gmax_hbm.at[pl.ds(base * ngrp, rpw * ngrp)], gm_v)
    iota = lax.iota(jnp.int32, 16)

    def row_body(r, carry):
        acc16, r16 = carry
        T, TI = plsc.sort_key_val(gm_v[pl.ds(r * ngrp, 16)], iota)
        for c in range(1, ngrp // 16):
            v = gm_v[pl.ds(r * ngrp + c * 16, 16)]
            sv, si = plsc.sort_key_val(v, iota + c * 16, descending=True)
            m = T >= sv
            T, TI = plsc.sort_key_val(jnp.where(m, T, sv),
                                      jnp.where(m, TI, si))
        TId = plsc.sort_key_val(T, TI, descending=True)[1]
        idx_v[...] = TId + (base + r) * ngrp
        pltpu.async_copy(hflat_hbm.at[idx_v.at[pl.ds(0, K)]],
                         gat_v.at[pl.ds(0, K)], sem).wait()
        g0 = gat_v[0, :]
        S = plsc.sort_key_val(g0, g0)[0]
        for c in range(1, K):
            gc = gat_v[c, :]
            vd = plsc.sort_key_val(gc, gc, descending=True)[0]
            Sm = jnp.maximum(S, vd)
            S = plsc.sort_key_val(Sm, Sm)[0]
        tot = jnp.sum(jnp.where(iota >= 16 - K, S, 0.0))
        acc16 = jnp.where(iota == r16, tot, acc16)

        @pl.when(r16 == 15)
        def _():
            res_v[pl.ds((r // 16) * 16, 16)] = acc16
        r16 = jnp.where(r16 == 15, 0, r16 + 1)
        return acc16, r16

    lax.fori_loop(0, rpw, row_body,
                  (jnp.zeros((16,), jnp.float32), jnp.int32(0)))
    pltpu.sync_copy(res_v, out_hbm.at[pl.ds(base, rpw)])


NCHUNK = 4


def kernel(x, W, b):
    Bm, D = x.shape
    N = W.shape[0]
    nj = N // N_BLK
    ngrp = N // GRP
    b2 = b.reshape(1, N)

    bc = Bm // NCHUNK
    nb = bc // B_BLK
    rpw = bc // NWORKERS

    mm_call = pl.pallas_call(
        functools.partial(_mm_body, n_blk=N_BLK),
        grid=(nj, nb),
        in_specs=[
            pl.BlockSpec((B_BLK, D), lambda j, i: (i, 0)),
            pl.BlockSpec((N_BLK, D), lambda j, i: (j, 0)),
            pl.BlockSpec((1, N_BLK), lambda j, i: (0, j)),
        ],
        out_specs=[
            pl.BlockSpec((B_BLK, N_BLK), lambda j, i: (i, j)),
            pl.BlockSpec((B_BLK, N_BLK // GRP), lambda j, i: (i, j)),
        ],
        out_shape=[
            jax.ShapeDtypeStruct((bc, N), jnp.float32),
            jax.ShapeDtypeStruct((bc, ngrp), jnp.float32),
        ],
        scratch_shapes=[pltpu.VMEM((1, N), jnp.float32),
                        pltpu.VMEM((N_BLK, N_BLK // GRP), jnp.float32)],
    )

    sc_sel = functools.partial(
        pl.kernel,
        out_type=jax.ShapeDtypeStruct((bc,), jnp.float32),
        mesh=plsc.VectorSubcoreMesh(core_axis_name="c", subcore_axis_name="s",
                                    num_cores=2, num_subcores=16),
        scratch_types=[
            pltpu.VMEM((rpw * ngrp,), jnp.float32),
            pltpu.VMEM((16,), jnp.int32),
            pltpu.VMEM((16, GRP), jnp.float32),
            pltpu.VMEM((rpw,), jnp.float32),
            pltpu.SemaphoreType.DMA,
        ],
        compiler_params=pltpu.CompilerParams(needs_layout_passes=False,
                                             use_tc_tiling_on_sc=False),
    )(functools.partial(_sc_body, rpw=rpw, ngrp=ngrp))

    parts = []
    for c in range(NCHUNK):
        xc = lax.slice_in_dim(x, c * bc, (c + 1) * bc, axis=0)
        h, gmax = mm_call(xc, W, b2)
        hflat = h.reshape(bc * ngrp, GRP)
        gflat = gmax.reshape(bc * ngrp)
        parts.append(sc_sel(gflat, hflat))
    per_inst = jnp.concatenate(parts)
    return jnp.mean(per_inst)


# speedup vs baseline: 1.1470x; 1.1470x over previous
import functools

import jax
import jax.numpy as jnp
from jax import lax
from jax.experimental import pallas as pl
from jax.experimental.pallas import tpu as pltpu
from jax.experimental.pallas import tpu_sc as plsc

K = 10
PENAL = 0.05

B_BLK = 256
N_BLK = 2048
GRP = 16
NWORKERS = 32


def _mm_body(x_ref, w_ref, b_ref, h_ref, g_ref, inv_ref, sel_ref, *, n_blk):
    j = pl.program_id(0)
    i = pl.program_id(1)

    w = w_ref[...]

    @pl.when(i == 0)
    def _():
        ssq = jnp.sum(w * w, axis=1)
        inv_ref[0, pl.ds(j * n_blk, n_blk)] = 1.0 / (jnp.sqrt(ssq) + 1e-12)

    @pl.when((i == 0) & (j == 0))
    def _():
        rows = lax.broadcasted_iota(jnp.int32, (n_blk, n_blk // GRP), 0)
        cols = lax.broadcasted_iota(jnp.int32, (n_blk, n_blk // GRP), 1)
        sel_ref[...] = (rows == GRP * cols + (GRP - 1)).astype(jnp.float32)

    inv = inv_ref[0, pl.ds(j * n_blk, n_blk)]
    pre = lax.dot_general(
        x_ref[...], w,
        dimension_numbers=(((1,), (1,)), ((), ())),
        preferred_element_type=jnp.float32,
    ) + b_ref[0, :][None, :]
    h = jnp.maximum(PENAL - jnp.abs(pre) * inv[None, :], 0.0)
    h_ref[...] = h
    m = h
    for s in (1, 2, 4, 8):
        m = jnp.maximum(m, pltpu.roll(m, s, 1))
    g_ref[...] = lax.dot_general(
        m, sel_ref[...],
        dimension_numbers=(((1,), (0,)), ((), ())),
        preferred_element_type=jnp.float32,
    )


def _sc_body(gmax_hbm, hflat_hbm, out_hbm, gm_v, idx_v, gat_v, res_v, sem,
             *, rpw, ngrp):
    nc = 2
    wid = lax.axis_index("s") * nc + lax.axis_index("c")
    base = wid * rpw
    pltpu.sync_copy(gmax_hbm.at[pl.ds(base * ngrp, rpw * ngrp)], gm_v)
    iota = lax.iota(jnp.int32, 16)

    def merge_kv(a, ai, b, bi):
        br = lax.rev(b, (0,))
        bri = lax.rev(bi, (0,))
        m = a >= br
        return plsc.sort_key_val(jnp.where(m, a, br), jnp.where(m, ai, bri))

    def merge_k(a, b):
        br = lax.rev(b, (0,))
        return plsc.sort_key_val(jnp.maximum(a, br), jnp.maximum(a, br))[0]

    def row_body(r, carry):
        acc16, r16 = carry
        lv = [plsc.sort_key_val(gm_v[pl.ds(r * ngrp + c * 16, 16)],
                                iota + c * 16)
              for c in range(ngrp // 16)]
        while len(lv) > 1:
            nxt = [merge_kv(lv[i][0], lv[i][1], lv[i + 1][0], lv[i + 1][1])
                   for i in range(0, len(lv) - 1, 2)]
            if len(lv) % 2:
                nxt.append(lv[-1])
            lv = nxt
        T, TI = lv[0]
        TId = plsc.sort_key_val(T, TI, descending=True)[1]
        idx_v[...] = TId + (base + r) * ngrp
        pltpu.async_copy(hflat_hbm.at[idx_v.at[pl.ds(0, K)]],
                         gat_v.at[pl.ds(0, K)], sem).wait()
        ls = []
        for c in range(K):
            gc = gat_v[c, :]
            ls.append(plsc.sort_key_val(gc, gc)[0])
        while len(ls) > 1:
            nxt = [merge_k(ls[i], ls[i + 1])
                   for i in range(0, len(ls) - 1, 2)]
            if len(ls) % 2:
                nxt.append(ls[-1])
            ls = nxt
        S = ls[0]
        tot = jnp.sum(jnp.where(iota >= 16 - K, S, 0.0))
        acc16 = jnp.where(iota == r16, tot, acc16)

        @pl.when(r16 == 15)
        def _():
            res_v[pl.ds((r // 16) * 16, 16)] = acc16
        r16 = jnp.where(r16 == 15, 0, r16 + 1)
        return acc16, r16

    lax.fori_loop(0, rpw, row_body,
                  (jnp.zeros((16,), jnp.float32), jnp.int32(0)))
    pltpu.sync_copy(res_v, out_hbm.at[pl.ds(base, rpw)])


NCHUNK = 1


def kernel(x, W, b):
    Bm, D = x.shape
    N = W.shape[0]
    nj = N // N_BLK
    ngrp = N // GRP
    b2 = b.reshape(1, N)

    bc = Bm // NCHUNK
    nb = bc // B_BLK
    rpw = bc // NWORKERS

    mm_call = pl.pallas_call(
        functools.partial(_mm_body, n_blk=N_BLK),
        grid=(nj, nb),
        in_specs=[
            pl.BlockSpec((B_BLK, D), lambda j, i: (i, 0)),
            pl.BlockSpec((N_BLK, D), lambda j, i: (j, 0)),
            pl.BlockSpec((1, N_BLK), lambda j, i: (0, j)),
        ],
        out_specs=[
            pl.BlockSpec((B_BLK, N_BLK), lambda j, i: (i, j)),
            pl.BlockSpec((B_BLK, N_BLK // GRP), lambda j, i: (i, j)),
        ],
        out_shape=[
            jax.ShapeDtypeStruct((bc, N), jnp.float32),
            jax.ShapeDtypeStruct((bc, ngrp), jnp.float32),
        ],
        scratch_shapes=[pltpu.VMEM((1, N), jnp.float32),
                        pltpu.VMEM((N_BLK, N_BLK // GRP), jnp.float32)],
    )

    sc_sel = functools.partial(
        pl.kernel,
        out_type=jax.ShapeDtypeStruct((bc,), jnp.float32),
        mesh=plsc.VectorSubcoreMesh(core_axis_name="c", subcore_axis_name="s",
                                    num_cores=2, num_subcores=16),
        scratch_types=[
            pltpu.VMEM((rpw * ngrp,), jnp.float32),
            pltpu.VMEM((16,), jnp.int32),
            pltpu.VMEM((16, GRP), jnp.float32),
            pltpu.VMEM((rpw,), jnp.float32),
            pltpu.SemaphoreType.DMA,
        ],
        compiler_params=pltpu.CompilerParams(needs_layout_passes=False,
                                             use_tc_tiling_on_sc=False),
    )(functools.partial(_sc_body, rpw=rpw, ngrp=ngrp))

    parts = []
    for c in range(NCHUNK):
        xc = lax.slice_in_dim(x, c * bc, (c + 1) * bc, axis=0)
        h, gmax = mm_call(xc, W, b2)
        hflat = h.reshape(bc * ngrp, GRP)
        gflat = gmax.reshape(bc * ngrp)
        parts.append(sc_sel(gflat, hflat))
    per_inst = jnp.concatenate(parts)
    return jnp.mean(per_inst)


# speedup vs baseline: 1.3080x; 1.1403x over previous
import functools

import jax
import jax.numpy as jnp
from jax import lax
from jax.experimental import pallas as pl
from jax.experimental.pallas import tpu as pltpu
from jax.experimental.pallas import tpu_sc as plsc

K = 10
PENAL = 0.05

B_BLK = 256
N_BLK = 2048
GRP = 16
NWORKERS = 32


def _mm_body(x_ref, w_ref, b_ref, h_ref, g_ref, inv_ref, sel_ref, *, n_blk):
    j = pl.program_id(0)
    i = pl.program_id(1)

    w = w_ref[...]

    @pl.when(i == 0)
    def _():
        ssq = jnp.sum(w * w, axis=1)
        inv_ref[0, pl.ds(j * n_blk, n_blk)] = 1.0 / (jnp.sqrt(ssq) + 1e-12)

    @pl.when((i == 0) & (j == 0))
    def _():
        rows = lax.broadcasted_iota(jnp.int32, (n_blk, n_blk // GRP), 0)
        cols = lax.broadcasted_iota(jnp.int32, (n_blk, n_blk // GRP), 1)
        sel_ref[...] = (rows == GRP * cols + (GRP - 1)).astype(jnp.float32)

    inv = inv_ref[0, pl.ds(j * n_blk, n_blk)]
    pre = lax.dot_general(
        x_ref[...], w,
        dimension_numbers=(((1,), (1,)), ((), ())),
        preferred_element_type=jnp.float32,
    ) + b_ref[0, :][None, :]
    h = jnp.maximum(PENAL - jnp.abs(pre) * inv[None, :], 0.0)
    h_ref[...] = h
    m = h
    for s in (1, 2, 4, 8):
        m = jnp.maximum(m, pltpu.roll(m, s, 1))
    g_ref[...] = lax.dot_general(
        m, sel_ref[...],
        dimension_numbers=(((1,), (0,)), ((), ())),
        preferred_element_type=jnp.float32,
    )


def _sc_body(gmax_hbm, hflat_hbm, out_hbm, gm_v, idx_v, gat_v, res_v,
             sem0, sem1, *, rpw, ngrp):
    nc = 2
    wid = lax.axis_index("s") * nc + lax.axis_index("c")
    base = wid * rpw
    pltpu.sync_copy(gmax_hbm.at[pl.ds(base * ngrp, rpw * ngrp)], gm_v)
    iota = lax.iota(jnp.int32, 16)

    def merge_kv(a, ai, b, bi):
        br = lax.rev(b, (0,))
        bri = lax.rev(bi, (0,))
        m = a >= br
        return plsc.sort_key_val(jnp.where(m, a, br), jnp.where(m, ai, bri))

    def merge_k(a, b):
        br = lax.rev(b, (0,))
        return plsc.sort_key_val(jnp.maximum(a, br), jnp.maximum(a, br))[0]

    def stage1(r):
        lv = [plsc.sort_key_val(gm_v[pl.ds(r * ngrp + c * 16, 16)],
                                iota + c * 16)
              for c in range(ngrp // 16)]
        while len(lv) > 1:
            nxt = [merge_kv(lv[i][0], lv[i][1], lv[i + 1][0], lv[i + 1][1])
                   for i in range(0, len(lv) - 1, 2)]
            if len(lv) % 2:
                nxt.append(lv[-1])
            lv = nxt
        T, TI = lv[0]
        TId = plsc.sort_key_val(T, TI, descending=True)[1]
        return TId + (base + r) * ngrp

    def issue(r):
        pr = r % 2
        idx_v[pl.ds(pr * 16, 16)] = stage1(r)

        @pl.when(pr == 0)
        def _():
            pltpu.async_copy(hflat_hbm.at[idx_v.at[pl.ds(0, K)]],
                             gat_v.at[0, pl.ds(0, K)], sem0)

        @pl.when(pr == 1)
        def _():
            pltpu.async_copy(hflat_hbm.at[idx_v.at[pl.ds(16, K)]],
                             gat_v.at[1, pl.ds(0, K)], sem1)

    def stage2(pr):
        ls = []
        for c in range(K):
            gc = gat_v[pr, c, :]
            ls.append(plsc.sort_key_val(gc, gc)[0])
        while len(ls) > 1:
            nxt = [merge_k(ls[i], ls[i + 1])
                   for i in range(0, len(ls) - 1, 2)]
            if len(ls) % 2:
                nxt.append(ls[-1])
            ls = nxt
        return jnp.sum(jnp.where(iota >= 16 - K, ls[0], 0.0))

    issue(jnp.int32(0))

    def row_body(r, carry):
        acc16, r16 = carry

        @pl.when(r + 1 < rpw)
        def _():
            issue(r + 1)

        @pl.when(r % 2 == 0)
        def _():
            pltpu.make_async_copy(hflat_hbm.at[idx_v.at[pl.ds(0, K)]],
                                  gat_v.at[0, pl.ds(0, K)], sem0).wait()

        @pl.when(r % 2 == 1)
        def _():
            pltpu.make_async_copy(hflat_hbm.at[idx_v.at[pl.ds(16, K)]],
                                  gat_v.at[1, pl.ds(0, K)], sem1).wait()

        tot = stage2(r % 2)
        acc16 = jnp.where(iota == r16, tot, acc16)

        @pl.when(r16 == 15)
        def _():
            res_v[pl.ds((r // 16) * 16, 16)] = acc16
        r16 = jnp.where(r16 == 15, 0, r16 + 1)
        return acc16, r16

    lax.fori_loop(0, rpw, row_body,
                  (jnp.zeros((16,), jnp.float32), jnp.int32(0)))
    pltpu.sync_copy(res_v, out_hbm.at[pl.ds(base, rpw)])


NCHUNK = 1


def kernel(x, W, b):
    Bm, D = x.shape
    N = W.shape[0]
    nj = N // N_BLK
    ngrp = N // GRP
    b2 = b.reshape(1, N)

    bc = Bm // NCHUNK
    nb = bc // B_BLK
    rpw = bc // NWORKERS

    mm_call = pl.pallas_call(
        functools.partial(_mm_body, n_blk=N_BLK),
        grid=(nj, nb),
        in_specs=[
            pl.BlockSpec((B_BLK, D), lambda j, i: (i, 0)),
            pl.BlockSpec((N_BLK, D), lambda j, i: (j, 0)),
            pl.BlockSpec((1, N_BLK), lambda j, i: (0, j)),
        ],
        out_specs=[
            pl.BlockSpec((B_BLK, N_BLK), lambda j, i: (i, j)),
            pl.BlockSpec((B_BLK, N_BLK // GRP), lambda j, i: (i, j)),
        ],
        out_shape=[
            jax.ShapeDtypeStruct((bc, N), jnp.float32),
            jax.ShapeDtypeStruct((bc, ngrp), jnp.float32),
        ],
        scratch_shapes=[pltpu.VMEM((1, N), jnp.float32),
                        pltpu.VMEM((N_BLK, N_BLK // GRP), jnp.float32)],
    )

    sc_sel = functools.partial(
        pl.kernel,
        out_type=jax.ShapeDtypeStruct((bc,), jnp.float32),
        mesh=plsc.VectorSubcoreMesh(core_axis_name="c", subcore_axis_name="s",
                                    num_cores=2, num_subcores=16),
        scratch_types=[
            pltpu.VMEM((rpw * ngrp,), jnp.float32),
            pltpu.VMEM((32,), jnp.int32),
            pltpu.VMEM((2, 16, GRP), jnp.float32),
            pltpu.VMEM((rpw,), jnp.float32),
            pltpu.SemaphoreType.DMA,
            pltpu.SemaphoreType.DMA,
        ],
        compiler_params=pltpu.CompilerParams(needs_layout_passes=False,
                                             use_tc_tiling_on_sc=False),
    )(functools.partial(_sc_body, rpw=rpw, ngrp=ngrp))

    parts = []
    for c in range(NCHUNK):
        xc = lax.slice_in_dim(x, c * bc, (c + 1) * bc, axis=0)
        h, gmax = mm_call(xc, W, b2)
        hflat = h.reshape(bc * ngrp, GRP)
        gflat = gmax.reshape(bc * ngrp)
        parts.append(sc_sel(gflat, hflat))
    per_inst = jnp.concatenate(parts)
    return jnp.mean(per_inst)


# speedup vs baseline: 1.4608x; 1.1168x over previous
import functools

import jax
import jax.numpy as jnp
from jax import lax
from jax.experimental import pallas as pl
from jax.experimental.pallas import tpu as pltpu
from jax.experimental.pallas import tpu_sc as plsc

K = 10
PENAL = 0.05

B_BLK = 256
N_BLK = 2048
GRP = 16
NWORKERS = 32


def _mm_body(x_ref, w_ref, b_ref, h_ref, g_ref, inv_ref, sel_ref, *, n_blk):
    j = pl.program_id(0)
    i = pl.program_id(1)

    w = w_ref[...]

    @pl.when(i == 0)
    def _():
        ssq = jnp.sum(w * w, axis=1)
        inv_ref[0, pl.ds(j * n_blk, n_blk)] = 1.0 / (jnp.sqrt(ssq) + 1e-12)

    @pl.when((i == 0) & (j == 0))
    def _():
        rows = lax.broadcasted_iota(jnp.int32, (n_blk, n_blk // GRP), 0)
        cols = lax.broadcasted_iota(jnp.int32, (n_blk, n_blk // GRP), 1)
        sel_ref[...] = (rows == GRP * cols + (GRP - 1)).astype(jnp.float32)

    inv = inv_ref[0, pl.ds(j * n_blk, n_blk)]
    pre = lax.dot_general(
        x_ref[...], w,
        dimension_numbers=(((1,), (1,)), ((), ())),
        preferred_element_type=jnp.float32,
    ) + b_ref[0, :][None, :]
    h = jnp.maximum(PENAL - jnp.abs(pre) * inv[None, :], 0.0)
    nb8 = h.shape[0] // 8
    h_ref[...] = jnp.swapaxes(h.reshape(nb8, 8, n_blk // 128, 128), 1, 2)
    m = h
    for s in (1, 2, 4, 8):
        m = jnp.maximum(m, pltpu.roll(m, s, 1))
    g_ref[...] = lax.dot_general(
        m, sel_ref[...],
        dimension_numbers=(((1,), (0,)), ((), ())),
        preferred_element_type=jnp.float32,
    )


def _sc_body(gmax_hbm, hflat_hbm, out_hbm, gm_v, idx_v, gat_v, res_v,
             sem0, sem1, *, rpw, ngrp):
    nc = 2
    wid = lax.axis_index("s") * nc + lax.axis_index("c")
    base = wid * rpw
    pltpu.sync_copy(gmax_hbm.at[pl.ds(base * ngrp, rpw * ngrp)], gm_v)
    iota = lax.iota(jnp.int32, 16)

    def merge_kv(a, ai, b, bi):
        br = lax.rev(b, (0,))
        bri = lax.rev(bi, (0,))
        m = a >= br
        return plsc.sort_key_val(jnp.where(m, a, br), jnp.where(m, ai, bri))

    def merge_k(a, b):
        br = lax.rev(b, (0,))
        return plsc.sort_key_val(jnp.maximum(a, br), jnp.maximum(a, br))[0]

    def stage1(r):
        lv = [plsc.sort_key_val(gm_v[pl.ds(r * ngrp + c * 16, 16)],
                                iota + c * 16)
              for c in range(ngrp // 16)]
        while len(lv) > 1:
            nxt = [merge_kv(lv[i][0], lv[i][1], lv[i + 1][0], lv[i + 1][1])
                   for i in range(0, len(lv) - 1, 2)]
            if len(lv) % 2:
                nxt.append(lv[-1])
            lv = nxt
        T, TI = lv[0]
        return plsc.sort_key_val(T, TI, descending=True)[1]

    def issue(r):
        gid = stage1(r)
        rr = base + r
        lbase = (rr // 8) * (8 * ngrp) + (rr % 8) * 8
        pr = r % 2
        idx_v[pl.ds(pr * 16, 16)] = (lbase
                                     + (gid >> 3) * 64 + (gid & 7))

        @pl.when(pr == 0)
        def _():
            pltpu.async_copy(hflat_hbm.at[idx_v.at[pl.ds(0, K)]],
                             gat_v.at[0, pl.ds(0, K)], sem0)

        @pl.when(pr == 1)
        def _():
            pltpu.async_copy(hflat_hbm.at[idx_v.at[pl.ds(16, K)]],
                             gat_v.at[1, pl.ds(0, K)], sem1)

    def stage2(pr):
        ls = []
        for c in range(K):
            gc = gat_v[pr, c, :]
            ls.append(plsc.sort_key_val(gc, gc)[0])
        while len(ls) > 1:
            nxt = [merge_k(ls[i], ls[i + 1])
                   for i in range(0, len(ls) - 1, 2)]
            if len(ls) % 2:
                nxt.append(ls[-1])
            ls = nxt
        return jnp.sum(jnp.where(iota >= 16 - K, ls[0], 0.0))

    issue(jnp.int32(0))

    def row_body(r, carry):
        acc16, r16 = carry

        @pl.when(r + 1 < rpw)
        def _():
            issue(r + 1)

        @pl.when(r % 2 == 0)
        def _():
            pltpu.make_async_copy(hflat_hbm.at[idx_v.at[pl.ds(0, K)]],
                                  gat_v.at[0, pl.ds(0, K)], sem0).wait()

        @pl.when(r % 2 == 1)
        def _():
            pltpu.make_async_copy(hflat_hbm.at[idx_v.at[pl.ds(16, K)]],
                                  gat_v.at[1, pl.ds(0, K)], sem1).wait()

        tot = stage2(r % 2)
        acc16 = jnp.where(iota == r16, tot, acc16)

        @pl.when(r16 == 15)
        def _():
            res_v[pl.ds((r // 16) * 16, 16)] = acc16
        r16 = jnp.where(r16 == 15, 0, r16 + 1)
        return acc16, r16

    lax.fori_loop(0, rpw, row_body,
                  (jnp.zeros((16,), jnp.float32), jnp.int32(0)))
    pltpu.sync_copy(res_v, out_hbm.at[pl.ds(base, rpw)])


NCHUNK = 1


def kernel(x, W, b):
    Bm, D = x.shape
    N = W.shape[0]
    nj = N // N_BLK
    ngrp = N // GRP
    b2 = b.reshape(1, N)

    bc = Bm // NCHUNK
    nb = bc // B_BLK
    rpw = bc // NWORKERS

    mm_call = pl.pallas_call(
        functools.partial(_mm_body, n_blk=N_BLK),
        grid=(nj, nb),
        in_specs=[
            pl.BlockSpec((B_BLK, D), lambda j, i: (i, 0)),
            pl.BlockSpec((N_BLK, D), lambda j, i: (j, 0)),
            pl.BlockSpec((1, N_BLK), lambda j, i: (0, j)),
        ],
        out_specs=[
            pl.BlockSpec((B_BLK // 8, N_BLK // 128, 8, 128),
                         lambda j, i: (i, j, 0, 0)),
            pl.BlockSpec((B_BLK, N_BLK // GRP), lambda j, i: (i, j)),
        ],
        out_shape=[
            jax.ShapeDtypeStruct((bc // 8, N // 128, 8, 128), jnp.float32),
            jax.ShapeDtypeStruct((bc, ngrp), jnp.float32),
        ],
        scratch_shapes=[pltpu.VMEM((1, N), jnp.float32),
                        pltpu.VMEM((N_BLK, N_BLK // GRP), jnp.float32)],
    )

    sc_sel = functools.partial(
        pl.kernel,
        out_type=jax.ShapeDtypeStruct((bc,), jnp.float32),
        mesh=plsc.VectorSubcoreMesh(core_axis_name="c", subcore_axis_name="s",
                                    num_cores=2, num_subcores=16),
        scratch_types=[
            pltpu.VMEM((rpw * ngrp,), jnp.float32),
            pltpu.VMEM((32,), jnp.int32),
            pltpu.VMEM((2, 16, GRP), jnp.float32),
            pltpu.VMEM((rpw,), jnp.float32),
            pltpu.SemaphoreType.DMA,
            pltpu.SemaphoreType.DMA,
        ],
        compiler_params=pltpu.CompilerParams(needs_layout_passes=False,
                                             use_tc_tiling_on_sc=False),
    )(functools.partial(_sc_body, rpw=rpw, ngrp=ngrp))

    parts = []
    for c in range(NCHUNK):
        xc = lax.slice_in_dim(x, c * bc, (c + 1) * bc, axis=0)
        h, gmax = mm_call(xc, W, b2)
        hflat = h.reshape(bc * ngrp, GRP)
        gflat = gmax.reshape(bc * ngrp)
        parts.append(sc_sel(gflat, hflat))
    per_inst = jnp.concatenate(parts)
    return jnp.mean(per_inst)
